# Initial kernel scaffold; baseline (speedup 1.0000x reference)
import jax
import jax.numpy as jnp
from jax.experimental import pallas as pl


def kernel(x, edge_index, W_msg, b_msg, W_node, b_node):
    raise NotImplementedError("write your pallas kernel here")


# speedup vs baseline: 5.3152x; 5.3152x over previous
import functools

import jax
import jax.numpy as jnp
from jax import lax
from jax.experimental import pallas as pl
from jax.experimental.pallas import tpu as pltpu
from jax.experimental.pallas import tpu_sc as plsc

NUM_STEPS = 3


def _msg_pre_body(h_ref, w1_ref, w2_ref, b_ref, a_ref, bb_ref):
    h = h_ref[...]
    a_ref[...] = jnp.dot(h, w1_ref[...], preferred_element_type=jnp.float32)
    bb_ref[...] = (
        jnp.dot(h, w2_ref[...], preferred_element_type=jnp.float32) + b_ref[...]
    )


def _node_body(h_ref, m_ref, w1_ref, w2_ref, b_ref, o_ref):
    m = m_ref[0] + m_ref[1]
    acc = jnp.dot(h_ref[...], w1_ref[...], preferred_element_type=jnp.float32)
    acc = acc + jnp.dot(m, w2_ref[...], preferred_element_type=jnp.float32)
    o_ref[...] = jnp.maximum(acc + b_ref[...], 0.0)


@functools.partial(jax.jit, static_argnames=("blk",))
def _msg_pre(h, w1, w2, b2d, blk):
    n, d = h.shape
    grid = (n // blk,)
    return pl.pallas_call(
        _msg_pre_body,
        grid=grid,
        in_specs=[
            pl.BlockSpec((blk, d), lambda i: (i, 0)),
            pl.BlockSpec((d, d), lambda i: (0, 0)),
            pl.BlockSpec((d, d), lambda i: (0, 0)),
            pl.BlockSpec((1, d), lambda i: (0, 0)),
        ],
        out_specs=[
            pl.BlockSpec((blk, d), lambda i: (i, 0)),
            pl.BlockSpec((blk, d), lambda i: (i, 0)),
        ],
        out_shape=[jax.ShapeDtypeStruct((n, d), jnp.float32)] * 2,
    )(h, w1, w2, b2d)


@functools.partial(jax.jit, static_argnames=("blk",))
def _node_update(h, m2, w1, w2, b2d, blk):
    n, d = h.shape
    grid = (n // blk,)
    return pl.pallas_call(
        _node_body,
        grid=grid,
        in_specs=[
            pl.BlockSpec((blk, d), lambda i: (i, 0)),
            pl.BlockSpec((2, blk, d), lambda i: (0, i, 0)),
            pl.BlockSpec((d, d), lambda i: (0, 0)),
            pl.BlockSpec((d, d), lambda i: (0, 0)),
            pl.BlockSpec((1, d), lambda i: (0, 0)),
        ],
        out_specs=pl.BlockSpec((blk, d), lambda i: (i, 0)),
        out_shape=jax.ShapeDtypeStruct((n, d), jnp.float32),
    )(h, m2, w1, w2, b2d)


_NC = 2
_NS = 16
_LANES = 16
_ECHK = 80


def _make_sc_edge(n_nodes, n_edges, d):
    nw = _NC * _NS
    per_w = n_edges // nw
    n_chunks = per_w // _ECHK
    vecs_per_row = d // _LANES
    row_chk = _ECHK
    n_row_chunks = n_nodes // row_chk
    max_rc_per_sub = -(-n_row_chunks // _NS)

    mesh = plsc.VectorSubcoreMesh(core_axis_name="c", subcore_axis_name="s")

    @functools.partial(
        pl.kernel,
        mesh=mesh,
        out_type=jax.ShapeDtypeStruct((_NC, n_nodes, d), jnp.float32),
        scratch_types=[
            pltpu.VMEM((_ECHK,), jnp.int32),
            pltpu.VMEM((_ECHK,), jnp.int32),
            pltpu.VMEM((_ECHK, d), jnp.float32),
            pltpu.VMEM((_ECHK, d), jnp.float32),
            pltpu.VMEM_SHARED((n_nodes, d), jnp.float32),
            pltpu.SemaphoreType.DMA,
            pltpu.SemaphoreType.DMA,
        ],
    )
    def sc_edge(a_hbm, b_hbm, src_hbm, dst_hbm, out_hbm,
                idx_s, idx_d, buf_a, buf_b, m_sh, sem_a, sem_b):
        c = lax.axis_index("c")
        s = lax.axis_index("s")
        wid = s * _NC + c

        def zrow(r, carry):
            for j in range(vecs_per_row):
                buf_a[r, pl.ds(j * _LANES, _LANES)] = jnp.zeros(
                    (_LANES,), jnp.float32)
            return carry
        lax.fori_loop(0, _ECHK, zrow, 0)
        for i in range(max_rc_per_sub):
            chunk = i * _NS + s

            @pl.when(chunk < n_row_chunks)
            def _():
                pltpu.sync_copy(buf_a, m_sh.at[pl.ds(chunk * row_chk, row_chk)])
        plsc.subcore_barrier()

        def chunk_body(i, carry):
            base = wid * per_w + i * _ECHK
            pltpu.sync_copy(src_hbm.at[pl.ds(base, _ECHK)], idx_s)
            pltpu.sync_copy(dst_hbm.at[pl.ds(base, _ECHK)], idx_d)
            cp_a = pltpu.async_copy(a_hbm.at[idx_s], buf_a, sem_a)
            cp_b = pltpu.async_copy(b_hbm.at[idx_d], buf_b, sem_b)
            cp_a.wait()
            cp_b.wait()

            def row(r, rc):
                for j in range(vecs_per_row):
                    sl = pl.ds(j * _LANES, _LANES)
                    av = buf_a[r, sl]
                    bv = buf_b[r, sl]
                    buf_a[r, sl] = jnp.maximum(av + bv, 0.0)
                return rc
            lax.fori_loop(0, _ECHK, row, 0)
            pltpu.sync_copy(buf_a, m_sh.at[idx_d], add=True)
            return carry
        lax.fori_loop(0, n_chunks, chunk_body, 0)
        plsc.subcore_barrier()

        for i in range(max_rc_per_sub):
            chunk = i * _NS + s

            @pl.when(chunk < n_row_chunks)
            def _():
                sl = pl.ds(chunk * row_chk, row_chk)
                pltpu.sync_copy(m_sh.at[sl], out_hbm.at[c, sl])

    return sc_edge


def kernel(x, edge_index, W_msg, b_msg, W_node, b_node):
    n, d = x.shape
    e = edge_index.shape[1]
    src = edge_index[0].astype(jnp.int32)
    dst = edge_index[1].astype(jnp.int32)
    w1 = W_msg[:d]
    w2 = W_msg[d:]
    wn1 = W_node[:d]
    wn2 = W_node[d:]
    bm = b_msg.reshape(1, d)
    bn = b_node.reshape(1, d)
    blk = 1000 if n % 1000 == 0 else n

    sc_edge = _make_sc_edge(n, e, d)

    h = x
    for _ in range(NUM_STEPS):
        a, b = _msg_pre(h, w1, w2, bm, blk=blk)
        m2 = sc_edge(a, b, src, dst)
        h = _node_update(h, m2, wn1, wn2, bn, blk=blk)
    return h


# speedup vs baseline: 6.8541x; 1.2895x over previous
import functools

import jax
import jax.numpy as jnp
from jax import lax
from jax.experimental import pallas as pl
from jax.experimental.pallas import tpu as pltpu
from jax.experimental.pallas import tpu_sc as plsc

NUM_STEPS = 3


def _msg_pre_body(h_ref, w1_ref, w2_ref, b_ref, a_ref, bb_ref):
    h = h_ref[...]
    a_ref[...] = jnp.dot(h, w1_ref[...], preferred_element_type=jnp.float32)
    bb_ref[...] = (
        jnp.dot(h, w2_ref[...], preferred_element_type=jnp.float32) + b_ref[...]
    )


def _node_body(h_ref, m_ref, w1_ref, w2_ref, b_ref, o_ref):
    m = m_ref[0] + m_ref[1]
    acc = jnp.dot(h_ref[...], w1_ref[...], preferred_element_type=jnp.float32)
    acc = acc + jnp.dot(m, w2_ref[...], preferred_element_type=jnp.float32)
    o_ref[...] = jnp.maximum(acc + b_ref[...], 0.0)


@functools.partial(jax.jit, static_argnames=("blk",))
def _msg_pre(h, w1, w2, b2d, blk):
    n, d = h.shape
    grid = (n // blk,)
    return pl.pallas_call(
        _msg_pre_body,
        grid=grid,
        in_specs=[
            pl.BlockSpec((blk, d), lambda i: (i, 0)),
            pl.BlockSpec((d, d), lambda i: (0, 0)),
            pl.BlockSpec((d, d), lambda i: (0, 0)),
            pl.BlockSpec((1, d), lambda i: (0, 0)),
        ],
        out_specs=[
            pl.BlockSpec((blk, d), lambda i: (i, 0)),
            pl.BlockSpec((blk, d), lambda i: (i, 0)),
        ],
        out_shape=[jax.ShapeDtypeStruct((n, d), jnp.float32)] * 2,
    )(h, w1, w2, b2d)


@functools.partial(jax.jit, static_argnames=("blk",))
def _node_update(h, m2, w1, w2, b2d, blk):
    n, d = h.shape
    grid = (n // blk,)
    return pl.pallas_call(
        _node_body,
        grid=grid,
        in_specs=[
            pl.BlockSpec((blk, d), lambda i: (i, 0)),
            pl.BlockSpec((2, blk, d), lambda i: (0, i, 0)),
            pl.BlockSpec((d, d), lambda i: (0, 0)),
            pl.BlockSpec((d, d), lambda i: (0, 0)),
            pl.BlockSpec((1, d), lambda i: (0, 0)),
        ],
        out_specs=pl.BlockSpec((blk, d), lambda i: (i, 0)),
        out_shape=jax.ShapeDtypeStruct((n, d), jnp.float32),
    )(h, m2, w1, w2, b2d)


_NC = 2
_NS = 16
_LANES = 16
_ECHK = 80


def _make_sc_edge(n_nodes, n_edges, d):
    nw = _NC * _NS
    per_w = n_edges // nw
    n_chunks = per_w // _ECHK
    n_pairs = n_chunks // 2
    vecs_per_row = d // _LANES
    row_chk = _ECHK
    n_row_chunks = n_nodes // row_chk
    max_rc_per_sub = -(-n_row_chunks // _NS)

    mesh = plsc.VectorSubcoreMesh(core_axis_name="c", subcore_axis_name="s")

    @functools.partial(
        pl.kernel,
        mesh=mesh,
        out_type=jax.ShapeDtypeStruct((_NC, n_nodes, d), jnp.float32),
        scratch_types=[
            pltpu.VMEM((2, _ECHK), jnp.int32),
            pltpu.VMEM((2, _ECHK), jnp.int32),
            pltpu.VMEM((_ECHK, d), jnp.float32),
            pltpu.VMEM((_ECHK, d), jnp.float32),
            pltpu.VMEM((_ECHK, d), jnp.float32),
            pltpu.VMEM((_ECHK, d), jnp.float32),
            pltpu.VMEM_SHARED((n_nodes, d), jnp.float32),
            pltpu.SemaphoreType.DMA,
            pltpu.SemaphoreType.DMA,
            pltpu.SemaphoreType.DMA,
            pltpu.SemaphoreType.DMA,
            pltpu.SemaphoreType.DMA,
            pltpu.SemaphoreType.DMA,
        ],
    )
    def sc_edge(a_hbm, b_hbm, src_hbm, dst_hbm, out_hbm,
                idx_s2, idx_d2, buf_a0, buf_b0, buf_a1, buf_b1, m_sh,
                sem_i0, sem_i1, sem_a0, sem_b0, sem_a1, sem_b1):
        c = lax.axis_index("c")
        s = lax.axis_index("s")
        wid = s * _NC + c
        w_base = wid * per_w
        bufs = ((buf_a0, buf_b0, sem_a0, sem_b0),
                (buf_a1, buf_b1, sem_a1, sem_b1))
        isems = (sem_i0, sem_i1)

        def zrow(r, carry):
            for j in range(vecs_per_row):
                buf_a0[r, pl.ds(j * _LANES, _LANES)] = jnp.zeros(
                    (_LANES,), jnp.float32)
            return carry
        lax.fori_loop(0, _ECHK, zrow, 0)
        for i in range(max_rc_per_sub):
            chunk = i * _NS + s

            @pl.when(chunk < n_row_chunks)
            def _():
                pltpu.sync_copy(buf_a0,
                                m_sh.at[pl.ds(chunk * row_chk, row_chk)])
        plsc.subcore_barrier()

        def issue_idx(i, p):
            base = w_base + i * _ECHK
            pltpu.async_copy(src_hbm.at[pl.ds(base, _ECHK)],
                             idx_s2.at[p], isems[p])
            pltpu.async_copy(dst_hbm.at[pl.ds(base, _ECHK)],
                             idx_d2.at[p], isems[p])

        def wait_idx(p):
            pltpu.make_async_copy(src_hbm.at[pl.ds(0, _ECHK)],
                                  idx_s2.at[p], isems[p]).wait()
            pltpu.make_async_copy(dst_hbm.at[pl.ds(0, _ECHK)],
                                  idx_d2.at[p], isems[p]).wait()

        def issue_gather(p):
            ba, bb, sa, sb = bufs[p]
            pltpu.async_copy(a_hbm.at[idx_s2.at[p]], ba, sa)
            pltpu.async_copy(b_hbm.at[idx_d2.at[p]], bb, sb)

        def wait_gather(p):
            ba, bb, sa, sb = bufs[p]
            pltpu.make_async_copy(a_hbm.at[idx_s2.at[p]], ba, sa).wait()
            pltpu.make_async_copy(b_hbm.at[idx_d2.at[p]], bb, sb).wait()

        def compute_scatter(p):
            ba, bb, _, _ = bufs[p]

            def row(r, rc):
                for j in range(vecs_per_row):
                    sl = pl.ds(j * _LANES, _LANES)
                    ba[r, sl] = jnp.maximum(ba[r, sl] + bb[r, sl], 0.0)
                return rc
            lax.fori_loop(0, _ECHK, row, 0)
            pltpu.sync_copy(ba, m_sh.at[idx_d2.at[p]], add=True)

        issue_idx(0, 0)
        issue_idx(1, 1)
        wait_idx(0)
        issue_gather(0)

        def pair_body(k, carry):
            wait_gather(0)
            compute_scatter(0)
            issue_idx(2 * k + 2, 0)
            wait_idx(1)
            issue_gather(1)
            wait_gather(1)
            compute_scatter(1)

            @pl.when(2 * k + 3 < n_chunks)
            def _():
                issue_idx(2 * k + 3, 1)
            wait_idx(0)
            issue_gather(0)
            return carry
        lax.fori_loop(0, n_pairs, pair_body, 0)
        wait_gather(0)
        compute_scatter(0)
        plsc.subcore_barrier()

        for i in range(max_rc_per_sub):
            chunk = i * _NS + s

            @pl.when(chunk < n_row_chunks)
            def _():
                sl = pl.ds(chunk * row_chk, row_chk)
                pltpu.sync_copy(m_sh.at[sl], out_hbm.at[c, sl])

    return sc_edge


def kernel(x, edge_index, W_msg, b_msg, W_node, b_node):
    n, d = x.shape
    e = edge_index.shape[1]
    src = edge_index[0].astype(jnp.int32)
    dst = edge_index[1].astype(jnp.int32)
    w1 = W_msg[:d]
    w2 = W_msg[d:]
    wn1 = W_node[:d]
    wn2 = W_node[d:]
    bm = b_msg.reshape(1, d)
    bn = b_node.reshape(1, d)
    blk = 1000 if n % 1000 == 0 else n

    sc_edge = _make_sc_edge(n, e, d)

    h = x
    for _ in range(NUM_STEPS):
        a, b = _msg_pre(h, w1, w2, bm, blk=blk)
        m2 = sc_edge(a, b, src, dst)
        h = _node_update(h, m2, wn1, wn2, bn, blk=blk)
    return h


# speedup vs baseline: 7.8675x; 1.1478x over previous
import functools

import jax
import jax.numpy as jnp
from jax import lax
from jax.experimental import pallas as pl
from jax.experimental.pallas import tpu as pltpu
from jax.experimental.pallas import tpu_sc as plsc

NUM_STEPS = 3


def _msg_pre_body(h_ref, w1_ref, w2_ref, b_ref, a_ref, bb_ref):
    h = h_ref[...]
    a_ref[...] = jnp.dot(h, w1_ref[...], preferred_element_type=jnp.float32)
    bb_ref[...] = (
        jnp.dot(h, w2_ref[...], preferred_element_type=jnp.float32) + b_ref[...]
    )


def _node_body(h_ref, m_ref, w1_ref, w2_ref, b_ref, o_ref):
    m = m_ref[0] + m_ref[1]
    acc = jnp.dot(h_ref[...], w1_ref[...], preferred_element_type=jnp.float32)
    acc = acc + jnp.dot(m, w2_ref[...], preferred_element_type=jnp.float32)
    o_ref[...] = jnp.maximum(acc + b_ref[...], 0.0)


@functools.partial(jax.jit, static_argnames=("blk",))
def _msg_pre(h, w1, w2, b2d, blk):
    n, d = h.shape
    grid = (n // blk,)
    return pl.pallas_call(
        _msg_pre_body,
        grid=grid,
        in_specs=[
            pl.BlockSpec((blk, d), lambda i: (i, 0)),
            pl.BlockSpec((d, d), lambda i: (0, 0)),
            pl.BlockSpec((d, d), lambda i: (0, 0)),
            pl.BlockSpec((1, d), lambda i: (0, 0)),
        ],
        out_specs=[
            pl.BlockSpec((blk, d), lambda i: (i, 0)),
            pl.BlockSpec((blk, d), lambda i: (i, 0)),
        ],
        out_shape=[jax.ShapeDtypeStruct((n, d), jnp.float32)] * 2,
    )(h, w1, w2, b2d)


@functools.partial(jax.jit, static_argnames=("blk",))
def _node_update(h, m2, w1, w2, b2d, blk):
    n, d = h.shape
    grid = (n // blk,)
    return pl.pallas_call(
        _node_body,
        grid=grid,
        in_specs=[
            pl.BlockSpec((blk, d), lambda i: (i, 0)),
            pl.BlockSpec((2, blk, d), lambda i: (0, i, 0)),
            pl.BlockSpec((d, d), lambda i: (0, 0)),
            pl.BlockSpec((d, d), lambda i: (0, 0)),
            pl.BlockSpec((1, d), lambda i: (0, 0)),
        ],
        out_specs=pl.BlockSpec((blk, d), lambda i: (i, 0)),
        out_shape=jax.ShapeDtypeStruct((n, d), jnp.float32),
    )(h, m2, w1, w2, b2d)


_NC = 2
_NS = 16
_LANES = 16
_ECHK = 80


def _make_sc_edge(n_nodes, n_edges, d):
    nw = _NC * _NS
    per_w = n_edges // nw
    n_chunks = per_w // _ECHK
    n_pairs = n_chunks // 2
    vecs_per_row = d // _LANES
    row_chk = _ECHK
    n_row_chunks = n_nodes // row_chk
    max_rc_per_sub = -(-n_row_chunks // _NS)

    mesh = plsc.VectorSubcoreMesh(core_axis_name="c", subcore_axis_name="s")

    @functools.partial(
        pl.kernel,
        mesh=mesh,
        out_type=jax.ShapeDtypeStruct((_NC, n_nodes, d), jnp.float32),
        scratch_types=[
            pltpu.VMEM((2, _ECHK), jnp.int32),
            pltpu.VMEM((2, _ECHK), jnp.int32),
            pltpu.VMEM((_ECHK, d), jnp.float32),
            pltpu.VMEM((_ECHK, d), jnp.float32),
            pltpu.VMEM((_ECHK, d), jnp.float32),
            pltpu.VMEM((_ECHK, d), jnp.float32),
            pltpu.VMEM((2, _ECHK), jnp.int32),
            pltpu.VMEM_SHARED((n_nodes, d), jnp.float32),
            pltpu.SemaphoreType.DMA,
            pltpu.SemaphoreType.DMA,
            pltpu.SemaphoreType.DMA,
            pltpu.SemaphoreType.DMA,
            pltpu.SemaphoreType.DMA,
            pltpu.SemaphoreType.DMA,
            pltpu.SemaphoreType.DMA,
            pltpu.SemaphoreType.DMA,
        ],
    )
    def sc_edge(a_hbm, b_hbm, src_hbm, dst_hbm, out_hbm,
                idx_s2, idx_d2, buf_a0, buf_b0, buf_a1, buf_b1, sidx2, m_sh,
                sem_i0, sem_i1, sem_a0, sem_b0, sem_a1, sem_b1,
                sem_s0, sem_s1):
        c = lax.axis_index("c")
        s = lax.axis_index("s")
        wid = s * _NC + c
        w_base = wid * per_w
        bufs = ((buf_a0, buf_b0, sem_a0, sem_b0),
                (buf_a1, buf_b1, sem_a1, sem_b1))
        isems = (sem_i0, sem_i1)
        ssems = (sem_s0, sem_s1)

        def zrow(r, carry):
            for j in range(vecs_per_row):
                buf_a0[r, pl.ds(j * _LANES, _LANES)] = jnp.zeros(
                    (_LANES,), jnp.float32)
            return carry
        lax.fori_loop(0, _ECHK, zrow, 0)
        for i in range(max_rc_per_sub):
            chunk = i * _NS + s

            @pl.when(chunk < n_row_chunks)
            def _():
                pltpu.sync_copy(buf_a0,
                                m_sh.at[pl.ds(chunk * row_chk, row_chk)])
        plsc.subcore_barrier()

        def issue_idx(i, p):
            base = w_base + i * _ECHK
            pltpu.async_copy(src_hbm.at[pl.ds(base, _ECHK)],
                             idx_s2.at[p], isems[p])
            pltpu.async_copy(dst_hbm.at[pl.ds(base, _ECHK)],
                             idx_d2.at[p], isems[p])

        def wait_idx(p):
            pltpu.make_async_copy(src_hbm.at[pl.ds(0, _ECHK)],
                                  idx_s2.at[p], isems[p]).wait()
            pltpu.make_async_copy(dst_hbm.at[pl.ds(0, _ECHK)],
                                  idx_d2.at[p], isems[p]).wait()

        def issue_gather(p):
            ba, bb, sa, sb = bufs[p]
            pltpu.async_copy(a_hbm.at[idx_s2.at[p]], ba, sa)
            pltpu.async_copy(b_hbm.at[idx_d2.at[p]], bb, sb)

        def wait_gather(p):
            ba, bb, sa, sb = bufs[p]
            pltpu.make_async_copy(a_hbm.at[idx_s2.at[p]], ba, sa).wait()
            pltpu.make_async_copy(b_hbm.at[idx_d2.at[p]], bb, sb).wait()

        def compute(p):
            ba, bb, _, _ = bufs[p]
            for j in range(_ECHK // _LANES):
                sl = pl.ds(j * _LANES, _LANES)
                sidx2[p, sl] = idx_d2[p, sl]

            def row4(r4, rc):
                r0 = r4 * 4
                for u in range(4):
                    for j in range(vecs_per_row):
                        sl = pl.ds(j * _LANES, _LANES)
                        ba[r0 + u, sl] = jnp.maximum(
                            ba[r0 + u, sl] + bb[r0 + u, sl], 0.0)
                return rc
            lax.fori_loop(0, _ECHK // 4, row4, 0)

        def scatter_start(p):
            ba = bufs[p][0]
            pltpu.async_copy(ba, m_sh.at[sidx2.at[p]], ssems[p], add=True)

        def scatter_wait(p):
            ba = bufs[p][0]
            pltpu.make_async_copy(ba, m_sh.at[sidx2.at[p]], ssems[p]).wait()

        issue_idx(0, 0)
        issue_idx(1, 1)
        wait_idx(0)
        issue_gather(0)

        def pair_body(k, carry):
            wait_gather(0)
            compute(0)
            scatter_start(0)
            issue_idx(2 * k + 2, 0)
            wait_idx(1)

            @pl.when(k > 0)
            def _():
                scatter_wait(1)
            issue_gather(1)
            wait_gather(1)
            compute(1)
            scatter_start(1)

            @pl.when(2 * k + 3 < n_chunks)
            def _():
                issue_idx(2 * k + 3, 1)
            wait_idx(0)
            scatter_wait(0)
            issue_gather(0)
            return carry
        lax.fori_loop(0, n_pairs, pair_body, 0)
        wait_gather(0)
        compute(0)
        scatter_start(0)
        scatter_wait(1)
        scatter_wait(0)
        plsc.subcore_barrier()

        for i in range(max_rc_per_sub):
            chunk = i * _NS + s

            @pl.when(chunk < n_row_chunks)
            def _():
                sl = pl.ds(chunk * row_chk, row_chk)
                pltpu.sync_copy(m_sh.at[sl], out_hbm.at[c, sl])

    return sc_edge


def kernel(x, edge_index, W_msg, b_msg, W_node, b_node):
    n, d = x.shape
    e = edge_index.shape[1]
    src = edge_index[0].astype(jnp.int32)
    dst = edge_index[1].astype(jnp.int32)
    w1 = W_msg[:d]
    w2 = W_msg[d:]
    wn1 = W_node[:d]
    wn2 = W_node[d:]
    bm = b_msg.reshape(1, d)
    bn = b_node.reshape(1, d)
    blk = 1000 if n % 1000 == 0 else n

    sc_edge = _make_sc_edge(n, e, d)

    h = x
    for _ in range(NUM_STEPS):
        a, b = _msg_pre(h, w1, w2, bm, blk=blk)
        m2 = sc_edge(a, b, src, dst)
        h = _node_update(h, m2, wn1, wn2, bn, blk=blk)
    return h
